# speedup vs baseline: 15.4808x; 15.4808x over previous
import math

import jax
import jax.numpy as jnp
from jax import lax
from jax.experimental import pallas as pl
from jax.experimental.pallas import tpu as pltpu

_TEMP = 10000.0
_SCALE = 2.0 * math.pi
_EPS_POS = 1e-6
_EPS_LN = 1e-12


def _body(tv_ref, tq_ref, tt_col_ref, tt_row_ref, w_col_ref, b_col_ref,
          w_row_ref, b_row_ref, otv_ref, otq_ref, pos2d_ref, pos1d_ref):
    b = pl.program_id(0)
    C, HW = pos2d_ref.shape
    N = pos1d_ref.shape[0]
    H = 32
    W = HW // H

    @pl.when(b == 0)
    def _init():
        ci = lax.broadcasted_iota(jnp.int32, (C, HW), 0)
        hwi = lax.broadcasted_iota(jnp.int32, (C, HW), 1)
        h = (hwi // W + 1).astype(jnp.float32)
        w = (hwi % W + 1).astype(jnp.float32)
        half = C // 2
        is_y = ci < half
        embed = jnp.where(is_y,
                          h * (_SCALE / (H + _EPS_POS)),
                          w * (_SCALE / (W + _EPS_POS)))
        j = jnp.where(is_y, ci, ci - half)
        expo = (2.0 / half) * (j // 2).astype(jnp.float32)
        inv_dim_t = jnp.exp(expo * (-math.log(_TEMP)))
        ang = embed * inv_dim_t
        pos = jnp.where(j % 2 == 0, jnp.sin(ang), jnp.cos(ang))
        pos2d_ref[...] = pos + tt_col_ref[:, 1:2]

        ni = lax.broadcasted_iota(jnp.int32, (N, C), 0).astype(jnp.float32)
        cj = lax.broadcasted_iota(jnp.int32, (N, C), 1)
        expo1 = (2.0 / C) * (cj // 2).astype(jnp.float32)
        inv_dim_t1 = jnp.exp(expo1 * (-math.log(_TEMP)))
        ang1 = ni * inv_dim_t1
        pos1 = jnp.where(cj % 2 == 0, jnp.sin(ang1), jnp.cos(ang1))
        pos1d_ref[...] = pos1 + tt_row_ref[0:1, :]

    x = tv_ref[0] + pos2d_ref[...]
    mu = jnp.mean(x, axis=0, keepdims=True)
    xc = x - mu
    var = jnp.mean(xc * xc, axis=0, keepdims=True)
    otv_ref[0] = xc * lax.rsqrt(var + _EPS_LN) * w_col_ref[...] + b_col_ref[...]

    q = tq_ref[0] + pos1d_ref[...]
    mu1 = jnp.mean(q, axis=1, keepdims=True)
    qc = q - mu1
    var1 = jnp.mean(qc * qc, axis=1, keepdims=True)
    otq_ref[0] = qc * lax.rsqrt(var1 + _EPS_LN) * w_row_ref[...] + b_row_ref[...]


def kernel(input_tv, input_tq, tv_positions, tq_positions, token_type_table,
           ln_weight, ln_bias):
    B, C, H, W = input_tv.shape
    N = input_tq.shape[1]
    HW = H * W

    tv3 = input_tv.reshape(B, C, HW)
    tt_col = token_type_table.T
    w_col = ln_weight.reshape(C, 1)
    b_col = ln_bias.reshape(C, 1)
    w_row = ln_weight.reshape(1, C)
    b_row = ln_bias.reshape(1, C)

    full = lambda shape: pl.BlockSpec(shape, lambda b, _s=len(shape): (0,) * _s)

    otv, otq = pl.pallas_call(
        _body,
        grid=(B,),
        in_specs=[
            pl.BlockSpec((1, C, HW), lambda b: (b, 0, 0)),
            pl.BlockSpec((1, N, C), lambda b: (b, 0, 0)),
            full((C, 2)),
            full((2, C)),
            full((C, 1)),
            full((C, 1)),
            full((1, C)),
            full((1, C)),
        ],
        out_specs=[
            pl.BlockSpec((1, C, HW), lambda b: (b, 0, 0)),
            pl.BlockSpec((1, N, C), lambda b: (b, 0, 0)),
        ],
        out_shape=[
            jax.ShapeDtypeStruct((B, C, HW), jnp.float32),
            jax.ShapeDtypeStruct((B, N, C), jnp.float32),
        ],
        scratch_shapes=[
            pltpu.VMEM((C, HW), jnp.float32),
            pltpu.VMEM((N, C), jnp.float32),
        ],
        compiler_params=pltpu.CompilerParams(
            dimension_semantics=("arbitrary",),
        ),
    )(tv3, input_tq, tt_col, token_type_table, w_col, b_col, w_row, b_row)

    return otv.reshape(B, C, H, W), otq
